# Initial kernel scaffold; baseline (speedup 1.0000x reference)
import jax
import jax.numpy as jnp
from jax.experimental import pallas as pl


def kernel(inpt, W_router, w, b):
    raise NotImplementedError("write your pallas kernel here")


# speedup vs baseline: 1.9479x; 1.9479x over previous
import jax
import jax.numpy as jnp
from jax.experimental import pallas as pl

E = 8
D = 768
BLOCK = 2048


def _block_kernel(x_ref, wr_ref, w_ref, b_ref, o_ref):
    x = x_ref[...]
    logits = jnp.dot(x, wr_ref[...], preferred_element_type=jnp.float32)
    values = jnp.argmax(logits, axis=-1)
    eids = jax.lax.broadcasted_iota(jnp.int32, (x.shape[0], E), 1)
    onehot = (values[:, None] == eids).astype(jnp.float32)
    w_eff = jnp.dot(onehot, w_ref[...], preferred_element_type=jnp.float32)
    b_eff = jnp.dot(onehot, b_ref[...], preferred_element_type=jnp.float32)
    o_ref[...] = x * w_eff + b_eff


def kernel(inpt, W_router, w, b):
    n, d = inpt.shape
    grid = (n // BLOCK,)
    return pl.pallas_call(
        _block_kernel,
        grid=grid,
        in_specs=[
            pl.BlockSpec((BLOCK, d), lambda i: (i, 0)),
            pl.BlockSpec((d, E), lambda i: (0, 0)),
            pl.BlockSpec((E, d), lambda i: (0, 0)),
            pl.BlockSpec((E, d), lambda i: (0, 0)),
        ],
        out_specs=pl.BlockSpec((BLOCK, d), lambda i: (i, 0)),
        out_shape=jax.ShapeDtypeStruct((n, d), jnp.float32),
    )(inpt, W_router, w, b)
